# initial kernel scaffold (unmeasured)
import functools

import jax
import jax.numpy as jnp
from jax import lax
from jax.experimental import pallas as pl
from jax.experimental.pallas import tpu as pltpu

N_DEV = 32
B_PER = 512
D_MODEL = 256
H_PER = 512
B_GLOBAL = N_DEV * B_PER


def kernel(x, Win0, Wout0, Win1, Wout1, Win2, Wout2):
    def body(x_ref, win0_ref, wout0_ref, win1_ref, wout1_ref, win2_ref,
             wout2_ref, out_ref, xg_ref, pbuf_ref, stage_ref, wi_ref, wo_ref,
             ag_sems, rs_sems, send_sem):
        d = lax.axis_index("i")
        left = jnp.mod(d - 1, N_DEV)
        right = jnp.mod(d + 1, N_DEV)

        def chunk_rows(c):
            return pl.ds(c * B_PER, B_PER)

        barrier = pltpu.get_barrier_semaphore()
        for nbr in (left, right):
            pl.semaphore_signal(barrier, inc=1, device_id=(nbr,),
                                device_id_type=pl.DeviceIdType.MESH)
        pl.semaphore_wait(barrier, 2)

        xg_ref[chunk_rows(d), :] = x_ref[:, :].astype(jnp.bfloat16)

        def ring_allgather():
            for s in range(N_DEV - 1):
                c_send = jnp.mod(d - s, N_DEV)
                c_recv = jnp.mod(d - 1 - s, N_DEV)
                send = pltpu.make_async_remote_copy(
                    src_ref=xg_ref.at[chunk_rows(c_send)],
                    dst_ref=xg_ref.at[chunk_rows(c_send)],
                    send_sem=send_sem,
                    recv_sem=ag_sems.at[s],
                    device_id=(right,),
                    device_id_type=pl.DeviceIdType.MESH,
                )
                send.start()
                recv = pltpu.make_async_remote_copy(
                    src_ref=xg_ref.at[chunk_rows(c_recv)],
                    dst_ref=xg_ref.at[chunk_rows(c_recv)],
                    send_sem=send_sem,
                    recv_sem=ag_sems.at[s],
                    device_id=(right,),
                    device_id_type=pl.DeviceIdType.MESH,
                )
                recv.wait_recv()
                send.wait_send()

        ring_allgather()

        def compute_layer(win_ref, wout_ref):
            wi_ref[:, :] = win_ref[:, :].astype(jnp.bfloat16)
            wo_ref[:, :] = wout_ref[:, :].astype(jnp.bfloat16)

            def chunk_body(b, carry):
                xb = xg_ref[chunk_rows(b), :]
                h = jnp.dot(xb, wi_ref[:, :],
                            preferred_element_type=jnp.float32)
                h = jnp.maximum(h, 0.0).astype(jnp.bfloat16)
                p = jnp.dot(h, wo_ref[:, :],
                            preferred_element_type=jnp.float32)
                pbuf_ref[chunk_rows(b), :] = p.astype(jnp.bfloat16)
                return carry

            lax.fori_loop(0, N_DEV, chunk_body, 0)

        def ring_reduce_scatter():
            c0 = jnp.mod(d - 1, N_DEV)
            stage_ref[N_DEV - 1, :, :] = (
                pbuf_ref[chunk_rows(c0), :].astype(jnp.float32))
            for s in range(N_DEV - 1):
                src_slot = (N_DEV - 1) if s == 0 else (s - 1)
                send = pltpu.make_async_remote_copy(
                    src_ref=stage_ref.at[src_slot],
                    dst_ref=stage_ref.at[s],
                    send_sem=send_sem,
                    recv_sem=rs_sems.at[s],
                    device_id=(right,),
                    device_id_type=pl.DeviceIdType.MESH,
                )
                send.start()
                recv = pltpu.make_async_remote_copy(
                    src_ref=stage_ref.at[s],
                    dst_ref=stage_ref.at[s],
                    send_sem=send_sem,
                    recv_sem=rs_sems.at[s],
                    device_id=(right,),
                    device_id_type=pl.DeviceIdType.MESH,
                )
                recv.wait_recv()
                send.wait_send()
                if s < N_DEV - 2:
                    c = jnp.mod(d - 2 - s, N_DEV)
                    stage_ref[s, :, :] = (
                        stage_ref[s, :, :]
                        + pbuf_ref[chunk_rows(c), :].astype(jnp.float32))
            return (stage_ref[N_DEV - 2, :, :]
                    + pbuf_ref[chunk_rows(d), :].astype(jnp.float32))

        for l, (wi, wo) in enumerate(((win0_ref, wout0_ref),
                                      (win1_ref, wout1_ref),
                                      (win2_ref, wout2_ref))):
            compute_layer(wi, wo)
            fin = ring_reduce_scatter()
            if l < 2:
                xg_ref[chunk_rows(d), :] = fin.astype(jnp.bfloat16)
                ring_allgather()
            else:
                out_ref[:, :] = fin

        @functools.partial(pl.run_scoped,
                           second_barrier=pltpu.SemaphoreType.REGULAR)
        def _(second_barrier):
            for nbr in (left, right):
                pl.semaphore_signal(second_barrier, inc=1, device_id=(nbr,),
                                    device_id_type=pl.DeviceIdType.MESH)
            pl.semaphore_wait(second_barrier, 2)

    return pl.pallas_call(
        body,
        out_shape=jax.ShapeDtypeStruct((B_PER, D_MODEL), jnp.float32),
        in_specs=[pl.BlockSpec(memory_space=pltpu.VMEM)] * 7,
        out_specs=pl.BlockSpec(memory_space=pltpu.VMEM),
        scratch_shapes=[
            pltpu.VMEM((B_GLOBAL, D_MODEL), jnp.bfloat16),
            pltpu.VMEM((B_GLOBAL, D_MODEL), jnp.bfloat16),
            pltpu.VMEM((N_DEV, B_PER, D_MODEL), jnp.float32),
            pltpu.VMEM((D_MODEL, H_PER), jnp.bfloat16),
            pltpu.VMEM((H_PER, D_MODEL), jnp.bfloat16),
            pltpu.SemaphoreType.DMA((N_DEV - 1,)),
            pltpu.SemaphoreType.DMA((N_DEV - 1,)),
            pltpu.SemaphoreType.DMA,
        ],
        compiler_params=pltpu.CompilerParams(collective_id=0),
    )(x, Win0, Wout0, Win1, Wout1, Win2, Wout2)


# baseline (device time: 1205602 ns/iter reference)
import functools

import jax
import jax.numpy as jnp
from jax import lax
from jax.experimental import pallas as pl
from jax.experimental.pallas import tpu as pltpu

N_DEV = 32
B_PER = 512
D_MODEL = 256
H_PER = 512
B_GLOBAL = N_DEV * B_PER


def kernel(x, Win0, Wout0, Win1, Wout1, Win2, Wout2):
    def body(x_ref, win0_ref, wout0_ref, win1_ref, wout1_ref, win2_ref,
             wout2_ref, out_ref, xg_ref, pbuf_ref, stage_ref, comm_ref,
             wi_ref, wo_ref, ag_sems, rs_sems, send_sem):
        d = lax.axis_index("i")
        left = jnp.mod(d - 1, N_DEV)
        right = jnp.mod(d + 1, N_DEV)

        def chunk_rows(c):
            return pl.ds(c * B_PER, B_PER)

        barrier = pltpu.get_barrier_semaphore()
        for nbr in (left, right):
            pl.semaphore_signal(barrier, inc=1, device_id=(nbr,),
                                device_id_type=pl.DeviceIdType.MESH)
        pl.semaphore_wait(barrier, 2)

        xg_ref[chunk_rows(d), :] = x_ref[:, :].astype(jnp.bfloat16)

        def ring_allgather():
            comm_ref[N_DEV - 1, :, :] = xg_ref[chunk_rows(d), :]
            for s in range(N_DEV - 1):
                src_slot = (N_DEV - 1) if s == 0 else (s - 1)
                send = pltpu.make_async_remote_copy(
                    src_ref=comm_ref.at[src_slot],
                    dst_ref=comm_ref.at[s],
                    send_sem=send_sem,
                    recv_sem=ag_sems.at[s],
                    device_id=(right,),
                    device_id_type=pl.DeviceIdType.MESH,
                )
                send.start()
                send.wait_recv()
                send.wait_send()
                c_recv = jnp.mod(d - 1 - s, N_DEV)
                xg_ref[chunk_rows(c_recv), :] = comm_ref[s, :, :]

        STAGE = 9
        if STAGE >= 1:
            ring_allgather()

        def compute_layer(win_ref, wout_ref):
            wi_ref[:, :] = win_ref[:, :].astype(jnp.bfloat16)
            wo_ref[:, :] = wout_ref[:, :].astype(jnp.bfloat16)

            def chunk_body(b, carry):
                xb = xg_ref[chunk_rows(b), :]
                h = jnp.dot(xb, wi_ref[:, :],
                            preferred_element_type=jnp.float32)
                h = jnp.maximum(h, 0.0).astype(jnp.bfloat16)
                p = jnp.dot(h, wo_ref[:, :],
                            preferred_element_type=jnp.float32)
                pbuf_ref[chunk_rows(b), :] = p.astype(jnp.bfloat16)
                return carry

            lax.fori_loop(0, N_DEV, chunk_body, 0)

        def ring_reduce_scatter():
            c0 = jnp.mod(d - 1, N_DEV)
            stage_ref[N_DEV - 1, :, :] = (
                pbuf_ref[chunk_rows(c0), :].astype(jnp.float32))
            for s in range(N_DEV - 1):
                src_slot = (N_DEV - 1) if s == 0 else (s - 1)
                send = pltpu.make_async_remote_copy(
                    src_ref=stage_ref.at[src_slot],
                    dst_ref=stage_ref.at[s],
                    send_sem=send_sem,
                    recv_sem=rs_sems.at[s],
                    device_id=(right,),
                    device_id_type=pl.DeviceIdType.MESH,
                )
                send.start()
                recv = pltpu.make_async_remote_copy(
                    src_ref=stage_ref.at[s],
                    dst_ref=stage_ref.at[s],
                    send_sem=send_sem,
                    recv_sem=rs_sems.at[s],
                    device_id=(right,),
                    device_id_type=pl.DeviceIdType.MESH,
                )
                recv.wait_recv()
                send.wait_send()
                if s < N_DEV - 2:
                    c = jnp.mod(d - 2 - s, N_DEV)
                    stage_ref[s, :, :] = (
                        stage_ref[s, :, :]
                        + pbuf_ref[chunk_rows(c), :].astype(jnp.float32))
            return (stage_ref[N_DEV - 2, :, :]
                    + pbuf_ref[chunk_rows(d), :].astype(jnp.float32))

        if STAGE == 1:
            out_ref[:, :] = xg_ref[chunk_rows(d), :].astype(jnp.float32)
        elif STAGE == 2:
            compute_layer(win0_ref, wout0_ref)
            out_ref[:, :] = pbuf_ref[chunk_rows(d), :].astype(jnp.float32)
        elif STAGE == 3:
            compute_layer(win0_ref, wout0_ref)
            out_ref[:, :] = ring_reduce_scatter()
        elif STAGE in (4, 5):
            compute_layer(win0_ref, wout0_ref)
            fin = ring_reduce_scatter()
            xg_ref[chunk_rows(d), :] = fin.astype(jnp.bfloat16)
            ring_allgather()
            if STAGE == 4:
                out_ref[:, :] = xg_ref[chunk_rows(d), :].astype(jnp.float32)
            else:
                compute_layer(win1_ref, wout1_ref)
                out_ref[:, :] = ring_reduce_scatter()
        else:
            for l, (wi, wo) in enumerate(((win0_ref, wout0_ref),
                                          (win1_ref, wout1_ref),
                                          (win2_ref, wout2_ref))):
                compute_layer(wi, wo)
                fin = ring_reduce_scatter()
                if l < 2:
                    xg_ref[chunk_rows(d), :] = fin.astype(jnp.bfloat16)
                    ring_allgather()
                else:
                    out_ref[:, :] = fin

        @functools.partial(pl.run_scoped,
                           second_barrier=pltpu.SemaphoreType.REGULAR)
        def _(second_barrier):
            for nbr in (left, right):
                pl.semaphore_signal(second_barrier, inc=1, device_id=(nbr,),
                                    device_id_type=pl.DeviceIdType.MESH)
            pl.semaphore_wait(second_barrier, 2)

    return pl.pallas_call(
        body,
        out_shape=jax.ShapeDtypeStruct((B_PER, D_MODEL), jnp.float32),
        in_specs=[pl.BlockSpec(memory_space=pltpu.VMEM)] * 7,
        out_specs=pl.BlockSpec(memory_space=pltpu.VMEM),
        scratch_shapes=[
            pltpu.VMEM((B_GLOBAL, D_MODEL), jnp.bfloat16),
            pltpu.VMEM((B_GLOBAL, D_MODEL), jnp.bfloat16),
            pltpu.VMEM((N_DEV, B_PER, D_MODEL), jnp.float32),
            pltpu.VMEM((N_DEV, B_PER, D_MODEL), jnp.bfloat16),
            pltpu.VMEM((D_MODEL, H_PER), jnp.bfloat16),
            pltpu.VMEM((H_PER, D_MODEL), jnp.bfloat16),
            pltpu.SemaphoreType.DMA((N_DEV - 1,)),
            pltpu.SemaphoreType.DMA((N_DEV - 1,)),
            pltpu.SemaphoreType.DMA,
        ],
        compiler_params=pltpu.CompilerParams(
            collective_id=0, vmem_limit_bytes=100 * 1024 * 1024),
    )(x, Win0, Wout0, Win1, Wout1, Win2, Wout2)
